# baseline (speedup 1.0000x reference)
import functools

import jax
import jax.numpy as jnp
from jax import lax
from jax.experimental import pallas as pl
from jax.experimental.pallas import tpu as pltpu
from jax.experimental.pallas import tpu_sc as plsc

NNODES = 10000
NCLASSES = 64
NITER = 5
ALPHA = 0.9
BATCH = 1024
GAMMA = 0.5
D_FEAT = 128
NDIM = 128
N_EDGES = 320000

HALF = NCLASSES // 2
NS = 16
EPT = N_EDGES // NS
CHUNK = 1024
NCHUNK = 20
TAIL = EPT - (NCHUNK - 1) * CHUNK
GSUB = CHUNK // 128
TGSUB = TAIL // 128
TREM = TAIL - TGSUB * 128
NP_PAD = 10240
NPT = NP_PAD // NS
SUBB = 160

C1 = ALPHA * NNODES / (2.0 * BATCH)
C2 = 1.0 - ALPHA


def _sc_body(logit_hbm, ei_hbm, w_hbm, steps_hbm, za_hbm, zb_hbm,
             colv0, rowv0, wv0, rows0, colv1, rowv1, wv1, rows1,
             colv2, rowv2, wv2, semi2,
             lv, stepsv,
             semi0, semg0, sems0, semi1, semg1, sems1, acc_sh):
    c = lax.axis_index("c")
    s = lax.axis_index("s")
    nb = s * NPT
    zb = c * NP_PAD + nb
    ebase = s * EPT

    def on_z(fn):
        @pl.when(c == 0)
        def _a():
            fn(za_hbm)

        @pl.when(c == 1)
        def _b():
            fn(zb_hbm)

    idx = [(colv0, rowv0, wv0, semi0), (colv1, rowv1, wv1, semi1),
           (colv2, rowv2, wv2, semi2)]
    rows = [(rows0, semg0, sems0), (rows1, semg1, sems1)]

    zero16 = jnp.zeros((16,), jnp.float32)

    def nsub(t):
        return GSUB if t < NCHUNK - 1 else TGSUB

    def ntot(t):
        return CHUNK if t < NCHUNK - 1 else TAIL

    def fire_idx(t, p):
        colb, rowb, wb, sem = idx[p]
        eb = t * CHUNK
        n = ntot(t)
        pltpu.async_copy(ei_hbm.at[1, pl.ds(ebase + eb, n)],
                         colb.at[pl.ds(0, n)], sem)
        pltpu.async_copy(w_hbm.at[pl.ds(ebase + eb, n)],
                         wb.at[pl.ds(0, n)], sem)
        for g in range(nsub(t)):
            pltpu.async_copy(ei_hbm.at[0, pl.ds(ebase + eb + g * 128, 128)],
                             rowb.at[g], sem)
        if t == NCHUNK - 1 and TREM:
            pltpu.async_copy(
                ei_hbm.at[0, pl.ds(ebase + eb + TGSUB * 128, TREM)],
                rowb.at[TGSUB].at[pl.ds(0, TREM)], sem)

    def wait_idx(t, p):
        colb, rowb, wb, sem = idx[p]
        n = ntot(t)
        pltpu.make_async_copy(ei_hbm.at[1, pl.ds(0, n)],
                              colb.at[pl.ds(0, n)], sem).wait()
        pltpu.make_async_copy(w_hbm.at[pl.ds(0, n)],
                              wb.at[pl.ds(0, n)], sem).wait()
        for g in range(nsub(t)):
            pltpu.make_async_copy(ei_hbm.at[0, pl.ds(0, 128)],
                                  rowb.at[g], sem).wait()
        if t == NCHUNK - 1 and TREM:
            pltpu.make_async_copy(ei_hbm.at[0, pl.ds(0, TREM)],
                                  rowb.at[TGSUB].at[pl.ds(0, TREM)],
                                  sem).wait()

    def fire_gather(t, r, p):
        colb = idx[r][0]
        rowsb, semg, _ = rows[p]

        def go(zref):
            for g in range(nsub(t)):
                pltpu.async_copy(zref.at[colb.at[pl.ds(g * 128, 128)]],
                                 rowsb.at[pl.ds(g * 128, 128)], semg)
            if t == NCHUNK - 1 and TREM:
                pltpu.async_copy(zref.at[colb.at[pl.ds(TGSUB * 128, TREM)]],
                                 rowsb.at[pl.ds(TGSUB * 128, TREM)], semg)

        on_z(go)

    def wait_gather(t, p):
        rowsb, semg, _ = rows[p]
        n = ntot(t)
        pltpu.make_async_copy(za_hbm.at[pl.ds(0, n)],
                              rowsb.at[pl.ds(0, n)], semg).wait()

    def fire_scatter(t, r, p):
        rowb = idx[r][1]
        rowsb, _, sems = rows[p]
        for g in range(nsub(t)):
            pltpu.async_copy(rowsb.at[pl.ds(g * 128, 128)],
                             acc_sh.at[rowb.at[g]], sems, add=True)
        if t == NCHUNK - 1 and TREM:
            pltpu.async_copy(rowsb.at[pl.ds(TGSUB * 128, TREM)],
                             acc_sh.at[rowb.at[TGSUB].at[pl.ds(0, TREM)]],
                             sems, add=True)

    def wait_scatter(t, p):
        rowsb, _, sems = rows[p]
        n = ntot(t)
        pltpu.make_async_copy(rowsb.at[pl.ds(0, n)],
                              acc_sh.at[pl.ds(0, n)], sems).wait()

    def scale(t, r, p):
        wb_ref = idx[r][2]
        rowsb = rows[p][0]

        @plsc.parallel_loop(0, ntot(t) // 16, 1, unroll=4)
        def _scale(e16):
            wvec = wb_ref[pl.ds(e16 * 16, 16)]
            for k in range(16):
                e = e16 * 16 + k
                wbk = jnp.full((16,), wvec[k])
                r0 = rowsb[e, pl.ds(0, 16)]
                rowsb[e, pl.ds(0, 16)] = r0 * wbk
                r1 = rowsb[e, pl.ds(16, 16)]
                rowsb[e, pl.ds(16, 16)] = r1 * wbk

    pltpu.sync_copy(steps_hbm, stepsv)
    pltpu.sync_copy(logit_hbm.at[pl.ds(zb, NPT)], lv)
    on_z(lambda zref: pltpu.sync_copy(lv, zref.at[pl.ds(nb, NPT)]))

    @pl.loop(0, NPT * 2, unroll=8)
    def _zero_init(v):
        r = v >> 1
        q = (v & 1) * 16
        rows0[r, pl.ds(q, 16)] = zero16

    pltpu.sync_copy(rows0.at[pl.ds(0, NPT)], acc_sh.at[pl.ds(nb, NPT)])
    plsc.subcore_barrier()

    c1v = jnp.full((16,), C1, jnp.float32)
    c2v = jnp.full((16,), C2, jnp.float32)

    @pl.loop(0, NITER)
    def _iter(j):
        fire_idx(0, 0)
        fire_idx(1, 1)
        wait_idx(0, 0)
        fire_gather(0, 0, 0)
        for t in range(NCHUNK):
            p = t % 2
            q = 1 - p
            if t >= 1:
                wait_scatter(t - 1, q)
            if t + 2 < NCHUNK:
                fire_idx(t + 2, (t + 2) % 3)
            if t + 1 < NCHUNK:
                wait_idx(t + 1, (t + 1) % 3)
                fire_gather(t + 1, (t + 1) % 3, q)
            wait_gather(t, p)
            scale(t, t % 3, p)
            fire_scatter(t, t % 3, p)
        wait_scatter(NCHUNK - 1, (NCHUNK - 1) % 2)
        plsc.subcore_barrier()

        stepj = stepsv[pl.ds(j * 16, 16)]
        pltpu.async_copy(acc_sh.at[pl.ds(nb, NPT)],
                         rows0.at[pl.ds(0, NPT)], semg0)
        on_z(lambda zref: pltpu.async_copy(zref.at[pl.ds(nb, NPT)],
                                           rows1.at[pl.ds(0, NPT)], semg1))
        pltpu.make_async_copy(acc_sh.at[pl.ds(nb, NPT)],
                              rows0.at[pl.ds(0, NPT)], semg0).wait()
        pltpu.make_async_copy(za_hbm.at[pl.ds(0, NPT)],
                              rows1.at[pl.ds(0, NPT)], semg1).wait()

        @pl.loop(0, NPT * 2, unroll=8)
        def _upd(v):
            r = v >> 1
            qq = (v & 1) * 16
            z = rows1[r, pl.ds(qq, 16)]
            a = rows0[r, pl.ds(qq, 16)]
            l = lv[r, pl.ds(qq, 16)]
            rows1[r, pl.ds(qq, 16)] = z + stepj * (c1v * a - z + c2v * l)

        on_z(lambda zref: pltpu.sync_copy(rows1.at[pl.ds(0, NPT)],
                                          zref.at[pl.ds(nb, NPT)]))

        @pl.loop(0, NPT * 2, unroll=8)
        def _zero(v):
            r = v >> 1
            qq = (v & 1) * 16
            rows0[r, pl.ds(qq, 16)] = zero16

        pltpu.sync_copy(rows0.at[pl.ds(0, NPT)], acc_sh.at[pl.ds(nb, NPT)])
        plsc.subcore_barrier()


def _dense_body(x_ref, wm_ref, bm_ref, wo_ref, bo_ref, o_ref):
    h = jnp.dot(x_ref[...], wm_ref[...], preferred_element_type=jnp.float32)
    h = jnp.maximum(h + bm_ref[...], 0.0)
    o_ref[...] = (jnp.dot(h, wo_ref[0], preferred_element_type=jnp.float32)
                  + bo_ref[0])


def _softmax_body(za_ref, zb_ref, o_ref):
    x = jnp.concatenate([za_ref[:NNODES], zb_ref[:NNODES]], axis=1)
    m = jnp.max(x, axis=1, keepdims=True)
    e = jnp.exp(x - m)
    o_ref[...] = e / jnp.sum(e, axis=1, keepdims=True)


def kernel(Xr, edge_index, edge_weight, W_mean, b_mean, W_out, b_out, i):
    ei = (edge_index if edge_index.dtype == jnp.int32
          else edge_index.astype(jnp.int32))
    w = (edge_weight if edge_weight.dtype == jnp.float32
         else edge_weight.astype(jnp.float32))

    ii = jnp.asarray(i, jnp.float32)
    steps = (1.0 + ii + jnp.arange(NITER, dtype=jnp.float32)) ** (-GAMMA)
    steps16 = jnp.repeat(steps, 16)

    bm2 = b_mean.reshape(1, D_FEAT)
    wo3 = jnp.stack([W_out[:, :HALF], W_out[:, HALF:]])
    bo3 = b_out.reshape(2, 1, HALF)

    xr_pad = jnp.concatenate(
        [Xr, jnp.zeros((NP_PAD - NNODES, D_FEAT), jnp.float32)])
    logit = pl.pallas_call(
        _dense_body,
        grid=(2,),
        in_specs=[
            pl.BlockSpec((NP_PAD, D_FEAT), lambda h: (0, 0)),
            pl.BlockSpec((D_FEAT, NDIM), lambda h: (0, 0)),
            pl.BlockSpec((1, NDIM), lambda h: (0, 0)),
            pl.BlockSpec((1, NDIM, HALF), lambda h: (h, 0, 0)),
            pl.BlockSpec((1, 1, HALF), lambda h: (h, 0, 0)),
        ],
        out_specs=pl.BlockSpec((NP_PAD, HALF), lambda h: (h, 0)),
        out_shape=jax.ShapeDtypeStruct((2 * NP_PAD, HALF), jnp.float32),
    )(xr_pad, W_mean, bm2, wo3, bo3)

    mesh = plsc.VectorSubcoreMesh(core_axis_name="c", subcore_axis_name="s")
    zf = pl.kernel(
        _sc_body,
        out_type=(jax.ShapeDtypeStruct((NP_PAD, HALF), jnp.float32),
                  jax.ShapeDtypeStruct((NP_PAD, HALF), jnp.float32)),
        mesh=mesh,
        compiler_params=pltpu.CompilerParams(use_tc_tiling_on_sc=False),
        scratch_types=[
            pltpu.VMEM((CHUNK,), jnp.int32),
            pltpu.VMEM((GSUB, 128), jnp.int32),
            pltpu.VMEM((CHUNK,), jnp.float32),
            pltpu.VMEM((CHUNK, HALF), jnp.float32),
            pltpu.VMEM((CHUNK,), jnp.int32),
            pltpu.VMEM((GSUB, 128), jnp.int32),
            pltpu.VMEM((CHUNK,), jnp.float32),
            pltpu.VMEM((CHUNK, HALF), jnp.float32),
            pltpu.VMEM((CHUNK,), jnp.int32),
            pltpu.VMEM((GSUB, 128), jnp.int32),
            pltpu.VMEM((CHUNK,), jnp.float32),
            pltpu.SemaphoreType.DMA,
            pltpu.VMEM((NPT, HALF), jnp.float32),
            pltpu.VMEM((NITER * 16,), jnp.float32),
            pltpu.SemaphoreType.DMA,
            pltpu.SemaphoreType.DMA,
            pltpu.SemaphoreType.DMA,
            pltpu.SemaphoreType.DMA,
            pltpu.SemaphoreType.DMA,
            pltpu.SemaphoreType.DMA,
            pltpu.VMEM_SHARED((NP_PAD, HALF), jnp.float32),
        ],
    )(logit, ei, w, steps16)
    za, zb2 = zf


    out = pl.pallas_call(
        _softmax_body,
        grid=(1,),
        in_specs=[
            pl.BlockSpec((NP_PAD, HALF), lambda m: (0, 0)),
            pl.BlockSpec((NP_PAD, HALF), lambda m: (0, 0)),
        ],
        out_specs=pl.BlockSpec((NNODES, NCLASSES), lambda m: (0, 0)),
        out_shape=jax.ShapeDtypeStruct((NNODES, NCLASSES), jnp.float32),
    )(za, zb2)
    return out


# speedup vs baseline: 1.0161x; 1.0161x over previous
import functools

import jax
import jax.numpy as jnp
from jax import lax
from jax.experimental import pallas as pl
from jax.experimental.pallas import tpu as pltpu
from jax.experimental.pallas import tpu_sc as plsc

NNODES = 10000
NCLASSES = 64
NITER = 5
ALPHA = 0.9
BATCH = 1024
GAMMA = 0.5
D_FEAT = 128
NDIM = 128
N_EDGES = 320000

HALF = NCLASSES // 2
NS = 16
EPT = N_EDGES // NS
CHUNK = 1024
NCHUNK = 20
TAIL = EPT - (NCHUNK - 1) * CHUNK
GSUB = CHUNK // 128
TGSUB = TAIL // 128
TREM = TAIL - TGSUB * 128
NP_PAD = 10240
NPT = NP_PAD // NS
SUBB = 160

C1 = ALPHA * NNODES / (2.0 * BATCH)
C2 = 1.0 - ALPHA


def _sc_body(logit_hbm, ei_hbm, w_hbm, steps_hbm, za_hbm, zb_hbm,
             colv0, rowv0, wv0, rows0, colv1, rowv1, wv1, rows1,
             colv2, rowv2, wv2, semi2,
             lv, stepsv,
             semi0, semg0, sems0, semi1, semg1, sems1, acc_sh):
    c = lax.axis_index("c")
    s = lax.axis_index("s")
    nb = s * NPT
    zb = c * NP_PAD + nb
    ebase = s * EPT

    def on_z(fn):
        @pl.when(c == 0)
        def _a():
            fn(za_hbm)

        @pl.when(c == 1)
        def _b():
            fn(zb_hbm)

    idx = [(colv0, rowv0, wv0, semi0), (colv1, rowv1, wv1, semi1),
           (colv2, rowv2, wv2, semi2)]
    rows = [(rows0, semg0, sems0), (rows1, semg1, sems1)]

    zero16 = jnp.zeros((16,), jnp.float32)

    def nsub(t):
        return GSUB if t < NCHUNK - 1 else TGSUB

    def ntot(t):
        return CHUNK if t < NCHUNK - 1 else TAIL

    def fire_idx(t, p):
        colb, rowb, wb, sem = idx[p]
        eb = t * CHUNK
        n = ntot(t)
        pltpu.async_copy(ei_hbm.at[1, pl.ds(ebase + eb, n)],
                         colb.at[pl.ds(0, n)], sem)
        pltpu.async_copy(w_hbm.at[pl.ds(ebase + eb, n)],
                         wb.at[pl.ds(0, n)], sem)
        for g in range(nsub(t)):
            pltpu.async_copy(ei_hbm.at[0, pl.ds(ebase + eb + g * 128, 128)],
                             rowb.at[g], sem)
        if t == NCHUNK - 1 and TREM:
            pltpu.async_copy(
                ei_hbm.at[0, pl.ds(ebase + eb + TGSUB * 128, TREM)],
                rowb.at[TGSUB].at[pl.ds(0, TREM)], sem)

    def wait_idx(t, p):
        colb, rowb, wb, sem = idx[p]
        n = ntot(t)
        pltpu.make_async_copy(ei_hbm.at[1, pl.ds(0, n)],
                              colb.at[pl.ds(0, n)], sem).wait()
        pltpu.make_async_copy(w_hbm.at[pl.ds(0, n)],
                              wb.at[pl.ds(0, n)], sem).wait()
        for g in range(nsub(t)):
            pltpu.make_async_copy(ei_hbm.at[0, pl.ds(0, 128)],
                                  rowb.at[g], sem).wait()
        if t == NCHUNK - 1 and TREM:
            pltpu.make_async_copy(ei_hbm.at[0, pl.ds(0, TREM)],
                                  rowb.at[TGSUB].at[pl.ds(0, TREM)],
                                  sem).wait()

    def fire_gather(t, r, p):
        colb = idx[r][0]
        rowsb, semg, _ = rows[p]

        def go(zref):
            for g in range(nsub(t)):
                pltpu.async_copy(zref.at[colb.at[pl.ds(g * 128, 128)]],
                                 rowsb.at[pl.ds(g * 128, 128)], semg)
            if t == NCHUNK - 1 and TREM:
                pltpu.async_copy(zref.at[colb.at[pl.ds(TGSUB * 128, TREM)]],
                                 rowsb.at[pl.ds(TGSUB * 128, TREM)], semg)

        on_z(go)

    def wait_gather(t, p):
        rowsb, semg, _ = rows[p]
        n = ntot(t)
        pltpu.make_async_copy(za_hbm.at[pl.ds(0, n)],
                              rowsb.at[pl.ds(0, n)], semg).wait()

    def fire_scatter(t, r, p):
        rowb = idx[r][1]
        rowsb, _, sems = rows[p]
        for g in range(nsub(t)):
            pltpu.async_copy(rowsb.at[pl.ds(g * 128, 128)],
                             acc_sh.at[rowb.at[g]], sems, add=True)
        if t == NCHUNK - 1 and TREM:
            pltpu.async_copy(rowsb.at[pl.ds(TGSUB * 128, TREM)],
                             acc_sh.at[rowb.at[TGSUB].at[pl.ds(0, TREM)]],
                             sems, add=True)

    def wait_scatter(t, p):
        rowsb, _, sems = rows[p]
        n = ntot(t)
        pltpu.make_async_copy(rowsb.at[pl.ds(0, n)],
                              acc_sh.at[pl.ds(0, n)], sems).wait()

    def scale(t, r, p):
        wb_ref = idx[r][2]
        rowsb = rows[p][0]

        @plsc.parallel_loop(0, ntot(t) // 16, 1, unroll=2)
        def _scale(e16):
            wvec = wb_ref[pl.ds(e16 * 16, 16)]
            for k in range(16):
                e = e16 * 16 + k
                wbk = jnp.full((16,), wvec[k])
                r0 = rowsb[e, pl.ds(0, 16)]
                rowsb[e, pl.ds(0, 16)] = r0 * wbk
                r1 = rowsb[e, pl.ds(16, 16)]
                rowsb[e, pl.ds(16, 16)] = r1 * wbk

    pltpu.sync_copy(steps_hbm, stepsv)
    pltpu.sync_copy(logit_hbm.at[pl.ds(zb, NPT)], lv)
    on_z(lambda zref: pltpu.sync_copy(lv, zref.at[pl.ds(nb, NPT)]))

    @pl.loop(0, NPT * 2, unroll=8)
    def _zero_init(v):
        r = v >> 1
        q = (v & 1) * 16
        rows0[r, pl.ds(q, 16)] = zero16

    pltpu.sync_copy(rows0.at[pl.ds(0, NPT)], acc_sh.at[pl.ds(nb, NPT)])
    plsc.subcore_barrier()

    c1v = jnp.full((16,), C1, jnp.float32)
    c2v = jnp.full((16,), C2, jnp.float32)

    @pl.loop(0, NITER)
    def _iter(j):
        fire_idx(0, 0)
        fire_idx(1, 1)
        wait_idx(0, 0)
        fire_gather(0, 0, 0)
        for t in range(NCHUNK):
            p = t % 2
            q = 1 - p
            if t >= 1:
                wait_scatter(t - 1, q)
            if t + 2 < NCHUNK:
                fire_idx(t + 2, (t + 2) % 3)
            if t + 1 < NCHUNK:
                wait_idx(t + 1, (t + 1) % 3)
                fire_gather(t + 1, (t + 1) % 3, q)
            wait_gather(t, p)
            scale(t, t % 3, p)
            fire_scatter(t, t % 3, p)
        wait_scatter(NCHUNK - 1, (NCHUNK - 1) % 2)
        plsc.subcore_barrier()

        stepj = stepsv[pl.ds(j * 16, 16)]
        pltpu.async_copy(acc_sh.at[pl.ds(nb, NPT)],
                         rows0.at[pl.ds(0, NPT)], semg0)
        on_z(lambda zref: pltpu.async_copy(zref.at[pl.ds(nb, NPT)],
                                           rows1.at[pl.ds(0, NPT)], semg1))
        pltpu.make_async_copy(acc_sh.at[pl.ds(nb, NPT)],
                              rows0.at[pl.ds(0, NPT)], semg0).wait()
        pltpu.make_async_copy(za_hbm.at[pl.ds(0, NPT)],
                              rows1.at[pl.ds(0, NPT)], semg1).wait()

        @pl.loop(0, NPT * 2, unroll=8)
        def _upd(v):
            r = v >> 1
            qq = (v & 1) * 16
            z = rows1[r, pl.ds(qq, 16)]
            a = rows0[r, pl.ds(qq, 16)]
            l = lv[r, pl.ds(qq, 16)]
            rows1[r, pl.ds(qq, 16)] = z + stepj * (c1v * a - z + c2v * l)

        on_z(lambda zref: pltpu.sync_copy(rows1.at[pl.ds(0, NPT)],
                                          zref.at[pl.ds(nb, NPT)]))

        @pl.loop(0, NPT * 2, unroll=8)
        def _zero(v):
            r = v >> 1
            qq = (v & 1) * 16
            rows0[r, pl.ds(qq, 16)] = zero16

        pltpu.sync_copy(rows0.at[pl.ds(0, NPT)], acc_sh.at[pl.ds(nb, NPT)])
        plsc.subcore_barrier()


def _dense_body(x_ref, wm_ref, bm_ref, wo_ref, bo_ref, o_ref):
    h = jnp.dot(x_ref[...], wm_ref[...], preferred_element_type=jnp.float32)
    h = jnp.maximum(h + bm_ref[...], 0.0)
    o_ref[...] = (jnp.dot(h, wo_ref[0], preferred_element_type=jnp.float32)
                  + bo_ref[0])


def _softmax_body(za_ref, zb_ref, o_ref):
    x = jnp.concatenate([za_ref[:NNODES], zb_ref[:NNODES]], axis=1)
    m = jnp.max(x, axis=1, keepdims=True)
    e = jnp.exp(x - m)
    o_ref[...] = e / jnp.sum(e, axis=1, keepdims=True)


def kernel(Xr, edge_index, edge_weight, W_mean, b_mean, W_out, b_out, i):
    ei = (edge_index if edge_index.dtype == jnp.int32
          else edge_index.astype(jnp.int32))
    w = (edge_weight if edge_weight.dtype == jnp.float32
         else edge_weight.astype(jnp.float32))

    ii = jnp.asarray(i, jnp.float32)
    steps = (1.0 + ii + jnp.arange(NITER, dtype=jnp.float32)) ** (-GAMMA)
    steps16 = jnp.repeat(steps, 16)

    bm2 = b_mean.reshape(1, D_FEAT)
    wo3 = jnp.stack([W_out[:, :HALF], W_out[:, HALF:]])
    bo3 = b_out.reshape(2, 1, HALF)

    xr_pad = jnp.concatenate(
        [Xr, jnp.zeros((NP_PAD - NNODES, D_FEAT), jnp.float32)])
    logit = pl.pallas_call(
        _dense_body,
        grid=(2,),
        in_specs=[
            pl.BlockSpec((NP_PAD, D_FEAT), lambda h: (0, 0)),
            pl.BlockSpec((D_FEAT, NDIM), lambda h: (0, 0)),
            pl.BlockSpec((1, NDIM), lambda h: (0, 0)),
            pl.BlockSpec((1, NDIM, HALF), lambda h: (h, 0, 0)),
            pl.BlockSpec((1, 1, HALF), lambda h: (h, 0, 0)),
        ],
        out_specs=pl.BlockSpec((NP_PAD, HALF), lambda h: (h, 0)),
        out_shape=jax.ShapeDtypeStruct((2 * NP_PAD, HALF), jnp.float32),
    )(xr_pad, W_mean, bm2, wo3, bo3)

    mesh = plsc.VectorSubcoreMesh(core_axis_name="c", subcore_axis_name="s")
    zf = pl.kernel(
        _sc_body,
        out_type=(jax.ShapeDtypeStruct((NP_PAD, HALF), jnp.float32),
                  jax.ShapeDtypeStruct((NP_PAD, HALF), jnp.float32)),
        mesh=mesh,
        compiler_params=pltpu.CompilerParams(use_tc_tiling_on_sc=False),
        scratch_types=[
            pltpu.VMEM((CHUNK,), jnp.int32),
            pltpu.VMEM((GSUB, 128), jnp.int32),
            pltpu.VMEM((CHUNK,), jnp.float32),
            pltpu.VMEM((CHUNK, HALF), jnp.float32),
            pltpu.VMEM((CHUNK,), jnp.int32),
            pltpu.VMEM((GSUB, 128), jnp.int32),
            pltpu.VMEM((CHUNK,), jnp.float32),
            pltpu.VMEM((CHUNK, HALF), jnp.float32),
            pltpu.VMEM((CHUNK,), jnp.int32),
            pltpu.VMEM((GSUB, 128), jnp.int32),
            pltpu.VMEM((CHUNK,), jnp.float32),
            pltpu.SemaphoreType.DMA,
            pltpu.VMEM((NPT, HALF), jnp.float32),
            pltpu.VMEM((NITER * 16,), jnp.float32),
            pltpu.SemaphoreType.DMA,
            pltpu.SemaphoreType.DMA,
            pltpu.SemaphoreType.DMA,
            pltpu.SemaphoreType.DMA,
            pltpu.SemaphoreType.DMA,
            pltpu.SemaphoreType.DMA,
            pltpu.VMEM_SHARED((NP_PAD, HALF), jnp.float32),
        ],
    )(logit, ei, w, steps16)
    za, zb2 = zf


    out = pl.pallas_call(
        _softmax_body,
        grid=(1,),
        in_specs=[
            pl.BlockSpec((NP_PAD, HALF), lambda m: (0, 0)),
            pl.BlockSpec((NP_PAD, HALF), lambda m: (0, 0)),
        ],
        out_specs=pl.BlockSpec((NNODES, NCLASSES), lambda m: (0, 0)),
        out_shape=jax.ShapeDtypeStruct((NNODES, NCLASSES), jnp.float32),
    )(za, zb2)
    return out
